# baseline (speedup 1.0000x reference)
import jax
import jax.numpy as jnp
from jax import lax
from jax.experimental import pallas as pl
from jax.experimental.pallas import tpu as pltpu
from jax.experimental.pallas import tpu_sc as plsc

B = 32
H = 56
W = 56
NCH = 3
VOC = 1024
D = 128
RCH = 1
PCH = RCH * W
NCHUNK = H // RCH


def _sc_body(in_hbm, emb_hbm, out_hbm, pl0, pl1, pl2, *rest):
    oixs = rest[:6]
    st0, st1, gsm0, gsm1, osm0, osm1 = rest[6:]
    cid = lax.axis_index("c")
    sid = lax.axis_index("s")
    b = sid * 2 + cid
    planes = (pl0, pl1, pl2)

    for c in range(NCH):
        pltpu.sync_copy(in_hbm.at[b, c], planes[c])

    def prep_and_gather(it, ph, stage, gsm):
        h0 = (it * 2 + ph) * RCH
        for c in range(NCH):
            oix = oixs[ph * NCH + c]
            for r in range(RCH):
                for w0 in (0, 16, 32, 40):
                    iv = planes[c][h0 + r, pl.ds(w0, 16)]
                    oix[pl.ds(r * W + w0, 16)] = iv + c * VOC
        for c in range(NCH):
            pltpu.async_copy(
                emb_hbm.at[oixs[ph * NCH + c]],
                stage.at[:, pl.ds(c * D, D)],
                gsm)

    def wait_gathers(ph, stage, gsm):
        for c in range(NCH):
            pltpu.make_async_copy(
                emb_hbm.at[oixs[ph * NCH + c]],
                stage.at[:, pl.ds(c * D, D)],
                gsm).wait()

    def out_dst(chunk):
        return out_hbm.at[b, pl.ds(chunk * PCH, PCH)]

    def pair_body(it, carry):
        @pl.when(it > 0)
        def _drain_prev_odd():
            wait_gathers(1, st1, gsm1)
            pltpu.async_copy(st1, out_dst(it * 2 - 1), osm1)
            pltpu.make_async_copy(st0, out_dst(0), osm0).wait()

        prep_and_gather(it, 0, st0, gsm0)

        wait_gathers(0, st0, gsm0)
        pltpu.async_copy(st0, out_dst(it * 2), osm0)

        @pl.when(it > 0)
        def _wait_old_odd_out():
            pltpu.make_async_copy(st1, out_dst(0), osm1).wait()

        prep_and_gather(it, 1, st1, gsm1)
        return carry

    lax.fori_loop(0, NCHUNK // 2, pair_body, 0)

    wait_gathers(1, st1, gsm1)
    pltpu.async_copy(st1, out_dst(NCHUNK - 1), osm1)
    pltpu.make_async_copy(st0, out_dst(0), osm0).wait()
    pltpu.make_async_copy(st1, out_dst(0), osm1).wait()


@jax.jit
def _sc_call(in_t, embedding):
    mesh = plsc.VectorSubcoreMesh(core_axis_name="c", subcore_axis_name="s")
    f = pl.kernel(
        _sc_body,
        out_type=jax.ShapeDtypeStruct((B, H * W, NCH * D), jnp.float32),
        mesh=mesh,
        scratch_types=[
            pltpu.VMEM((H, W), jnp.int32),
            pltpu.VMEM((H, W), jnp.int32),
            pltpu.VMEM((H, W), jnp.int32),
        ] + [pltpu.VMEM((PCH,), jnp.int32)] * 6 + [
            pltpu.VMEM((PCH, NCH * D), jnp.float32),
            pltpu.VMEM((PCH, NCH * D), jnp.float32),
            pltpu.SemaphoreType.DMA,
            pltpu.SemaphoreType.DMA,
            pltpu.SemaphoreType.DMA,
            pltpu.SemaphoreType.DMA,
        ],
        compiler_params=pltpu.CompilerParams(needs_layout_passes=False),
    )
    return f(in_t, embedding)


def kernel(inputs, embedding):
    in_t = jnp.transpose(inputs, (0, 3, 1, 2))
    out = _sc_call(in_t, embedding)
    out4 = out.reshape(B, H, W, NCH * D)
    return jnp.transpose(out4, (0, 3, 1, 2))


# speedup vs baseline: 1.0559x; 1.0559x over previous
import jax
import jax.numpy as jnp
from jax import lax
from jax.experimental import pallas as pl
from jax.experimental.pallas import tpu as pltpu
from jax.experimental.pallas import tpu_sc as plsc

B = 32
H = 56
W = 56
NCH = 3
VOC = 1024
D = 128
RCH = 2
PCH = RCH * W
NCHUNK = H // RCH


def _sc_body(in_hbm, emb_hbm, out_hbm, pl0, pl1, pl2, *rest):
    oixs = rest[:6]
    st0, st1, gsm0, gsm1, osm0, osm1 = rest[6:]
    cid = lax.axis_index("c")
    sid = lax.axis_index("s")
    b = sid * 2 + cid
    planes = (pl0, pl1, pl2)

    for c in range(NCH):
        pltpu.sync_copy(in_hbm.at[b, c], planes[c])

    def prep_and_gather(it, ph, stage, gsm):
        h0 = (it * 2 + ph) * RCH
        for c in range(NCH):
            oix = oixs[ph * NCH + c]
            for r in range(RCH):
                for w0 in (0, 16, 32, 40):
                    iv = planes[c][h0 + r, pl.ds(w0, 16)]
                    oix[pl.ds(r * W + w0, 16)] = iv + c * VOC
        for c in range(NCH):
            pltpu.async_copy(
                emb_hbm.at[oixs[ph * NCH + c]],
                stage.at[:, pl.ds(c * D, D)],
                gsm)

    def wait_gathers(ph, stage, gsm):
        for c in range(NCH):
            pltpu.make_async_copy(
                emb_hbm.at[oixs[ph * NCH + c]],
                stage.at[:, pl.ds(c * D, D)],
                gsm).wait()

    def out_dst(chunk):
        return out_hbm.at[b, pl.ds(chunk * PCH, PCH)]

    def pair_body(it, carry):
        @pl.when(it > 0)
        def _drain_prev_odd():
            wait_gathers(1, st1, gsm1)
            pltpu.async_copy(st1, out_dst(it * 2 - 1), osm1)
            pltpu.make_async_copy(st0, out_dst(0), osm0).wait()

        prep_and_gather(it, 0, st0, gsm0)

        wait_gathers(0, st0, gsm0)
        pltpu.async_copy(st0, out_dst(it * 2), osm0)

        @pl.when(it > 0)
        def _wait_old_odd_out():
            pltpu.make_async_copy(st1, out_dst(0), osm1).wait()

        prep_and_gather(it, 1, st1, gsm1)
        return carry

    lax.fori_loop(0, NCHUNK // 2, pair_body, 0)

    wait_gathers(1, st1, gsm1)
    pltpu.async_copy(st1, out_dst(NCHUNK - 1), osm1)
    pltpu.make_async_copy(st0, out_dst(0), osm0).wait()
    pltpu.make_async_copy(st1, out_dst(0), osm1).wait()


@jax.jit
def _sc_call(in_t, embedding):
    mesh = plsc.VectorSubcoreMesh(core_axis_name="c", subcore_axis_name="s")
    f = pl.kernel(
        _sc_body,
        out_type=jax.ShapeDtypeStruct((B, H * W, NCH * D), jnp.float32),
        mesh=mesh,
        scratch_types=[
            pltpu.VMEM((H, W), jnp.int32),
            pltpu.VMEM((H, W), jnp.int32),
            pltpu.VMEM((H, W), jnp.int32),
        ] + [pltpu.VMEM((PCH,), jnp.int32)] * 6 + [
            pltpu.VMEM((PCH, NCH * D), jnp.float32),
            pltpu.VMEM((PCH, NCH * D), jnp.float32),
            pltpu.SemaphoreType.DMA,
            pltpu.SemaphoreType.DMA,
            pltpu.SemaphoreType.DMA,
            pltpu.SemaphoreType.DMA,
        ],
        compiler_params=pltpu.CompilerParams(needs_layout_passes=False),
    )
    return f(in_t, embedding)


def kernel(inputs, embedding):
    in_t = jnp.transpose(inputs, (0, 3, 1, 2))
    out = _sc_call(in_t, embedding)
    out4 = out.reshape(B, H, W, NCH * D)
    return jnp.transpose(out4, (0, 3, 1, 2))
